# Initial kernel scaffold; baseline (speedup 1.0000x reference)
import jax
import jax.numpy as jnp
from jax.experimental import pallas as pl


def kernel(input, C):
    raise NotImplementedError("write your pallas kernel here")


# speedup vs baseline: 1.7991x; 1.7991x over previous
import functools

import jax
import jax.numpy as jnp
from jax import lax
from jax.experimental import pallas as pl
from jax.experimental.pallas import tpu as pltpu
from jax.experimental.pallas import tpu_sc as plsc

BATCH = 16384
HIST = 50
D = 64
B = BATCH * HIST
NC = 2
NS = 16
NW = NC * NS
BPW = B // NW
CH = 512
NCHUNK = BPW // CH


def _gather_body(idx_hbm, table_hbm, out_hbm, idx_v, rows_v, sem):
  wid = lax.axis_index("s") * NC + lax.axis_index("c")
  base = wid * BPW

  def step(g, carry):
    off = base + g * CH
    pltpu.sync_copy(idx_hbm.at[pl.ds(off, CH)], idx_v)
    pltpu.async_copy(table_hbm.at[idx_v], rows_v, sem).wait()
    pltpu.sync_copy(rows_v, out_hbm.at[pl.ds(off, CH)])
    return carry

  lax.fori_loop(0, NCHUNK, step, 0)


@jax.jit
def _embed(idx_flat, table):
  mesh = plsc.VectorSubcoreMesh(core_axis_name="c", subcore_axis_name="s")
  f = pl.kernel(
      _gather_body,
      out_type=jax.ShapeDtypeStruct((B, D), jnp.float32),
      mesh=mesh,
      scratch_types=[
          pltpu.VMEM((CH,), jnp.int32),
          pltpu.VMEM((CH, D), jnp.float32),
          pltpu.SemaphoreType.DMA,
      ],
      compiler_params=pltpu.CompilerParams(use_tc_tiling_on_sc=False),
  )
  return f(idx_flat, table)


def kernel(input, C):
  idx_flat = input.reshape(B).astype(jnp.int32)
  out = _embed(idx_flat, C)
  return out.reshape(BATCH, HIST, D)


# speedup vs baseline: 1.8767x; 1.0431x over previous
import functools

import jax
import jax.numpy as jnp
from jax import lax
from jax.experimental import pallas as pl
from jax.experimental.pallas import tpu as pltpu
from jax.experimental.pallas import tpu_sc as plsc

BATCH = 16384
HIST = 50
D = 64
B = BATCH * HIST
NC = 2
NS = 16
NW = NC * NS
BPW = B // NW
CH = 256
NCHUNK = BPW // CH
NBUF = 4


def _gather_body(idx_hbm, table_hbm, out_hbm, idx_v, rows_v, sem_g, sem_o):
  wid = lax.axis_index("s") * NC + lax.axis_index("c")
  base = wid * BPW
  pltpu.sync_copy(idx_hbm.at[pl.ds(base, BPW)], idx_v)

  def gather(g, b):
    return pltpu.make_async_copy(
        table_hbm.at[idx_v.at[pl.ds(g * CH, CH)]], rows_v.at[b], sem_g.at[b])

  def store(g, b):
    return pltpu.make_async_copy(
        rows_v.at[b], out_hbm.at[pl.ds(base + g * CH, CH)], sem_o.at[b])

  for b in range(NBUF):
    gather(b, b).start()

  def outer(i, carry):
    g0 = i * NBUF
    for b in range(NBUF):
      g = g0 + b
      gather(g, b).wait()
      store(g, b).start()
      store(g, b).wait()

      @pl.when(i + 1 < NCHUNK // NBUF)
      def _():
        gather(g + NBUF, b).start()

    return carry

  lax.fori_loop(0, NCHUNK // NBUF, outer, 0)


@jax.jit
def _embed(idx_flat, table):
  mesh = plsc.VectorSubcoreMesh(core_axis_name="c", subcore_axis_name="s")
  f = pl.kernel(
      _gather_body,
      out_type=jax.ShapeDtypeStruct((B, D), jnp.float32),
      mesh=mesh,
      scratch_types=[
          pltpu.VMEM((BPW,), jnp.int32),
          pltpu.VMEM((NBUF, CH, D), jnp.float32),
          pltpu.SemaphoreType.DMA((NBUF,)),
          pltpu.SemaphoreType.DMA((NBUF,)),
      ],
      compiler_params=pltpu.CompilerParams(use_tc_tiling_on_sc=False),
  )
  return f(idx_flat, table)


def kernel(input, C):
  idx_flat = input.reshape(B).astype(jnp.int32)
  out = _embed(idx_flat, C)
  return out.reshape(BATCH, HIST, D)
